# Initial kernel scaffold; baseline (speedup 1.0000x reference)
import jax
import jax.numpy as jnp
from jax import lax
from jax.experimental import pallas as pl
from jax.experimental.pallas import tpu as pltpu
from jax.experimental.pallas import tpu_sc as plsc

VOCAB = 100000
SEQ_LEN = 200
EMBED = 128
BATCH = 1024
EPS = 1e-5

NC = 2
NS = 16
NW = NC * NS
N_ROWS = BATCH * SEQ_LEN
ROWS_PER_TILE = N_ROWS // NW
CHUNK = 128
K = ROWS_PER_TILE // CHUNK
LANES = 16
NVREG = EMBED // LANES


def _sc_body(ids_hbm, table_hbm, pos_hbm, gamma_hbm, beta_hbm, out_hbm,
             idx_v, pos_v, g_v, b_v, buf_a, buf_b,
             gsem_a, gsem_b, osem_a, osem_b):
    wid = lax.axis_index("s") * NC + lax.axis_index("c")
    base_row = wid * ROWS_PER_TILE
    id_row0 = wid * K

    pltpu.sync_copy(ids_hbm.at[pl.ds(id_row0, K)], idx_v)
    pltpu.sync_copy(pos_hbm, pos_v)
    pltpu.sync_copy(gamma_hbm, g_v)
    pltpu.sync_copy(beta_hbm, b_v)

    inv_d = jnp.float32(1.0 / EMBED)

    def compute_chunk(buf, k):
        bs = lax.rem(base_row + k * CHUNK, SEQ_LEN)

        @pl.loop(0, CHUNK, unroll=4)
        def _row(r):
            s0 = bs + r
            s = jnp.where(s0 >= SEQ_LEN, s0 - SEQ_LEN, s0)
            ts = []
            for j in range(NVREG):
                x = buf[r, pl.ds(j * LANES, LANES)]
                p = pos_v[s, pl.ds(j * LANES, LANES)]
                ts.append(x + p)
            sv = list(ts)
            qv = [t * t for t in ts]
            while len(sv) > 1:
                sv = [sv[i] + sv[i + 1] for i in range(0, len(sv), 2)]
                qv = [qv[i] + qv[i + 1] for i in range(0, len(qv), 2)]
            tot = jnp.sum(sv[0])
            tot2 = jnp.sum(qv[0])
            mean = tot * inv_d
            var = tot2 * inv_d - mean * mean
            xv = var + jnp.float32(EPS)
            iv = lax.bitcast_convert_type(xv, jnp.int32)
            iv = jnp.int32(0x5F3759DF) - lax.shift_right_arithmetic(iv, 1)
            y = lax.bitcast_convert_type(iv, jnp.float32)
            for _ in range(3):
                y = y * (jnp.float32(1.5) - jnp.float32(0.5) * xv * y * y)
            rstd = y
            for j in range(NVREG):
                g = g_v[pl.ds(j * LANES, LANES)]
                b = b_v[pl.ds(j * LANES, LANES)]
                buf[r, pl.ds(j * LANES, LANES)] = (ts[j] - mean) * rstd * g + b

    def fire_gather(k, buf, sem):
        pltpu.async_copy(table_hbm.at[idx_v.at[k]], buf, sem)

    def wait_gather(k, buf, sem):
        pltpu.make_async_copy(table_hbm.at[idx_v.at[k]], buf, sem).wait()

    def fire_scatter(k, buf, sem):
        pltpu.async_copy(buf, out_hbm.at[pl.ds(base_row + k * CHUNK, CHUNK)], sem)

    def wait_scatter(k, buf, sem):
        pltpu.make_async_copy(
            buf, out_hbm.at[pl.ds(base_row + k * CHUNK, CHUNK)], sem).wait()

    fire_gather(0, buf_a, gsem_a)
    fire_gather(1, buf_b, gsem_b)

    @pl.loop(0, K, step=2)
    def _chunk(k):
        wait_gather(k, buf_a, gsem_a)
        compute_chunk(buf_a, k)
        fire_scatter(k, buf_a, osem_a)

        wait_gather(k + 1, buf_b, gsem_b)
        compute_chunk(buf_b, k + 1)
        fire_scatter(k + 1, buf_b, osem_b)

        wait_scatter(k, buf_a, osem_a)

        @pl.when(k + 2 < K)
        def _():
            fire_gather(k + 2, buf_a, gsem_a)

        wait_scatter(k + 1, buf_b, osem_b)

        @pl.when(k + 3 < K)
        def _():
            fire_gather(k + 3, buf_b, gsem_b)


@jax.jit
def _run(ids2d, token_table, pos_table, gamma, beta):
    mesh = plsc.VectorSubcoreMesh(core_axis_name="c", subcore_axis_name="s",
                                  num_cores=NC, num_subcores=NS)
    f = pl.kernel(
        _sc_body,
        out_type=jax.ShapeDtypeStruct((N_ROWS, EMBED), jnp.float32),
        mesh=mesh,
        scratch_types=[
            pltpu.VMEM((K, CHUNK), jnp.int32),
            pltpu.VMEM((SEQ_LEN, EMBED), jnp.float32),
            pltpu.VMEM((EMBED,), jnp.float32),
            pltpu.VMEM((EMBED,), jnp.float32),
            pltpu.VMEM((CHUNK, EMBED), jnp.float32),
            pltpu.VMEM((CHUNK, EMBED), jnp.float32),
            pltpu.SemaphoreType.DMA,
            pltpu.SemaphoreType.DMA,
            pltpu.SemaphoreType.DMA,
            pltpu.SemaphoreType.DMA,
        ],
    )
    return f(ids2d, token_table, pos_table, gamma, beta)


def kernel(input_ids, token_table, pos_table, gamma, beta):
    ids2d = jnp.reshape(input_ids.astype(jnp.int32), (N_ROWS // CHUNK, CHUNK))
    out = _run(ids2d, token_table, pos_table, gamma, beta)
    return out.reshape(BATCH, SEQ_LEN, EMBED)


# speedup vs baseline: 1.5516x; 1.5516x over previous
import jax
import jax.numpy as jnp
from jax import lax
from jax.experimental import pallas as pl
from jax.experimental.pallas import tpu as pltpu
from jax.experimental.pallas import tpu_sc as plsc

VOCAB = 100000
SEQ_LEN = 200
EMBED = 128
BATCH = 1024
EPS = 1e-5

NC = 2
NS = 16
NW = NC * NS
N_ROWS = BATCH * SEQ_LEN
ROWS_PER_TILE = N_ROWS // NW
CHUNK = 128
K = ROWS_PER_TILE // CHUNK
LANES = 16
NVREG = EMBED // LANES


def _sc_body(ids_hbm, table_hbm, pos_hbm, gamma_hbm, beta_hbm, out_hbm,
             idx_v, pos_v, g_v, b_v, buf_a, buf_b,
             gsem_a, gsem_b, osem_a, osem_b):
    wid = lax.axis_index("s") * NC + lax.axis_index("c")
    base_row = wid * ROWS_PER_TILE

    pltpu.sync_copy(ids_hbm.at[wid], idx_v)
    pltpu.sync_copy(pos_hbm, pos_v)
    pltpu.sync_copy(gamma_hbm, g_v)
    pltpu.sync_copy(beta_hbm, b_v)

    inv_d = jnp.float32(1.0 / EMBED)
    lane_iota = lax.iota(jnp.int32, LANES)
    perms = [lax.bitwise_xor(lane_iota, jnp.int32(d)) for d in (8, 4, 2, 1)]

    def lane_sum2(a, b):
        for p in perms:
            a = a + a.at[p].get(mode="promise_in_bounds")
            b = b + b.at[p].get(mode="promise_in_bounds")
        return a, b

    def compute_chunk(buf, k):
        bs = lax.rem(base_row + k * CHUNK, SEQ_LEN)

        @pl.loop(0, CHUNK, unroll=4)
        def _row(r):
            s0 = bs + r
            s = jnp.where(s0 >= SEQ_LEN, s0 - SEQ_LEN, s0)
            ts = []
            for j in range(NVREG):
                x = buf[r, pl.ds(j * LANES, LANES)]
                p = pos_v[s, pl.ds(j * LANES, LANES)]
                ts.append(x + p)
            sv = list(ts)
            qv = [t * t for t in ts]
            while len(sv) > 1:
                sv = [sv[i] + sv[i + 1] for i in range(0, len(sv), 2)]
                qv = [qv[i] + qv[i + 1] for i in range(0, len(qv), 2)]
            tot, tot2 = lane_sum2(sv[0], qv[0])
            mean = tot * inv_d
            var = tot2 * inv_d - mean * mean
            xv = var + jnp.float32(EPS)
            iv = lax.bitcast_convert_type(xv, jnp.int32)
            iv = jnp.int32(0x5F3759DF) - lax.shift_right_arithmetic(iv, 1)
            y = lax.bitcast_convert_type(iv, jnp.float32)
            for _ in range(3):
                y = y * (jnp.float32(1.5) - jnp.float32(0.5) * xv * y * y)
            rstd = y
            for j in range(NVREG):
                g = g_v[pl.ds(j * LANES, LANES)]
                b = b_v[pl.ds(j * LANES, LANES)]
                buf[r, pl.ds(j * LANES, LANES)] = (ts[j] - mean) * rstd * g + b

    def fire_gather(k, buf, sem):
        pltpu.async_copy(table_hbm.at[idx_v.at[k]], buf, sem)

    def wait_gather(k, buf, sem):
        pltpu.make_async_copy(table_hbm.at[idx_v.at[k]], buf, sem).wait()

    def fire_scatter(k, buf, sem):
        pltpu.async_copy(buf, out_hbm.at[pl.ds(base_row + k * CHUNK, CHUNK)], sem)

    def wait_scatter(k, buf, sem):
        pltpu.make_async_copy(
            buf, out_hbm.at[pl.ds(base_row + k * CHUNK, CHUNK)], sem).wait()

    fire_gather(0, buf_a, gsem_a)
    fire_gather(1, buf_b, gsem_b)

    @pl.loop(0, K, step=2)
    def _chunk(k):
        wait_gather(k, buf_a, gsem_a)
        compute_chunk(buf_a, k)
        fire_scatter(k, buf_a, osem_a)

        wait_gather(k + 1, buf_b, gsem_b)
        compute_chunk(buf_b, k + 1)
        fire_scatter(k + 1, buf_b, osem_b)

        wait_scatter(k, buf_a, osem_a)

        @pl.when(k + 2 < K)
        def _():
            fire_gather(k + 2, buf_a, gsem_a)

        wait_scatter(k + 1, buf_b, osem_b)

        @pl.when(k + 3 < K)
        def _():
            fire_gather(k + 3, buf_b, gsem_b)


@jax.jit
def _run(ids2d, token_table, pos_table, gamma, beta):
    mesh = plsc.VectorSubcoreMesh(core_axis_name="c", subcore_axis_name="s",
                                  num_cores=NC, num_subcores=NS)
    f = pl.kernel(
        _sc_body,
        out_type=jax.ShapeDtypeStruct((N_ROWS, EMBED), jnp.float32),
        mesh=mesh,
        scratch_types=[
            pltpu.VMEM((K, CHUNK), jnp.int32),
            pltpu.VMEM((SEQ_LEN, EMBED), jnp.float32),
            pltpu.VMEM((EMBED,), jnp.float32),
            pltpu.VMEM((EMBED,), jnp.float32),
            pltpu.VMEM((CHUNK, EMBED), jnp.float32),
            pltpu.VMEM((CHUNK, EMBED), jnp.float32),
            pltpu.SemaphoreType.DMA,
            pltpu.SemaphoreType.DMA,
            pltpu.SemaphoreType.DMA,
            pltpu.SemaphoreType.DMA,
        ],
    )
    return f(ids2d, token_table, pos_table, gamma, beta)


def kernel(input_ids, token_table, pos_table, gamma, beta):
    ids3d = jnp.reshape(input_ids.astype(jnp.int32), (NW, K, CHUNK))
    out = _run(ids3d, token_table, pos_table, gamma, beta)
    return out.reshape(BATCH, SEQ_LEN, EMBED)


# speedup vs baseline: 3.9979x; 2.5766x over previous
import jax
import jax.numpy as jnp
from jax import lax
from jax.experimental import pallas as pl
from jax.experimental.pallas import tpu as pltpu
from jax.experimental.pallas import tpu_sc as plsc

VOCAB = 100000
SEQ_LEN = 200
EMBED = 128
BATCH = 1024
EPS = 1e-5

NC = 2
NS = 16
NW = NC * NS
N_ROWS = BATCH * SEQ_LEN
ROWS_PER_TILE = N_ROWS // NW
CHUNK = 128
K = ROWS_PER_TILE // CHUNK

TC_B = 8


def _sc_gather_body(ids_hbm, table_hbm, out_hbm,
                    idx_v, buf_a, buf_b, gsem_a, gsem_b, osem_a, osem_b):
    wid = lax.axis_index("s") * NC + lax.axis_index("c")
    base_row = wid * ROWS_PER_TILE

    pltpu.sync_copy(ids_hbm.at[wid], idx_v)

    def fire_gather(k, buf, sem):
        pltpu.async_copy(table_hbm.at[idx_v.at[k]], buf, sem)

    def wait_gather(k, buf, sem):
        pltpu.make_async_copy(table_hbm.at[idx_v.at[k]], buf, sem).wait()

    def fire_scatter(k, buf, sem):
        pltpu.async_copy(buf, out_hbm.at[pl.ds(base_row + k * CHUNK, CHUNK)], sem)

    def wait_scatter(k, buf, sem):
        pltpu.make_async_copy(
            buf, out_hbm.at[pl.ds(base_row + k * CHUNK, CHUNK)], sem).wait()

    fire_gather(0, buf_a, gsem_a)
    fire_gather(1, buf_b, gsem_b)

    @pl.loop(0, K, step=2)
    def _chunk(k):
        wait_gather(k, buf_a, gsem_a)
        fire_scatter(k, buf_a, osem_a)

        wait_gather(k + 1, buf_b, gsem_b)
        fire_scatter(k + 1, buf_b, osem_b)

        wait_scatter(k, buf_a, osem_a)

        @pl.when(k + 2 < K)
        def _():
            fire_gather(k + 2, buf_a, gsem_a)

        wait_scatter(k + 1, buf_b, osem_b)

        @pl.when(k + 3 < K)
        def _():
            fire_gather(k + 3, buf_b, gsem_b)


def _tc_ln_body(x_ref, pos_ref, g_ref, b_ref, o_ref):
    t = x_ref[...] + pos_ref[...][None, :, :]
    mean = jnp.mean(t, axis=-1, keepdims=True)
    c = t - mean
    var = jnp.mean(c * c, axis=-1, keepdims=True)
    rstd = lax.rsqrt(var + EPS)
    o_ref[...] = c * rstd * g_ref[...] + b_ref[...]


@jax.jit
def _run(ids3d, token_table, pos_table, gamma, beta):
    mesh = plsc.VectorSubcoreMesh(core_axis_name="c", subcore_axis_name="s",
                                  num_cores=NC, num_subcores=NS)
    gathered = pl.kernel(
        _sc_gather_body,
        out_type=jax.ShapeDtypeStruct((N_ROWS, EMBED), jnp.float32),
        mesh=mesh,
        scratch_types=[
            pltpu.VMEM((K, CHUNK), jnp.int32),
            pltpu.VMEM((CHUNK, EMBED), jnp.float32),
            pltpu.VMEM((CHUNK, EMBED), jnp.float32),
            pltpu.SemaphoreType.DMA,
            pltpu.SemaphoreType.DMA,
            pltpu.SemaphoreType.DMA,
            pltpu.SemaphoreType.DMA,
        ],
    )(ids3d, token_table)

    x = gathered.reshape(BATCH, SEQ_LEN, EMBED)
    out = pl.pallas_call(
        _tc_ln_body,
        out_shape=jax.ShapeDtypeStruct((BATCH, SEQ_LEN, EMBED), jnp.float32),
        grid=(BATCH // TC_B,),
        in_specs=[
            pl.BlockSpec((TC_B, SEQ_LEN, EMBED), lambda i: (i, 0, 0)),
            pl.BlockSpec((SEQ_LEN, EMBED), lambda i: (0, 0)),
            pl.BlockSpec((EMBED,), lambda i: (0,)),
            pl.BlockSpec((EMBED,), lambda i: (0,)),
        ],
        out_specs=pl.BlockSpec((TC_B, SEQ_LEN, EMBED), lambda i: (i, 0, 0)),
    )(x, pos_table, gamma, beta)
    return out


def kernel(input_ids, token_table, pos_table, gamma, beta):
    ids3d = jnp.reshape(input_ids.astype(jnp.int32), (NW, K, CHUNK))
    return _run(ids3d, token_table, pos_table, gamma, beta)


# speedup vs baseline: 5.0758x; 1.2696x over previous
import jax
import jax.numpy as jnp
from jax import lax
from jax.experimental import pallas as pl
from jax.experimental.pallas import tpu as pltpu
from jax.experimental.pallas import tpu_sc as plsc

VOCAB = 100000
SEQ_LEN = 200
EMBED = 128
BATCH = 1024
EPS = 1e-5

NC = 2
NS = 16
NW = NC * NS
N_ROWS = BATCH * SEQ_LEN
ROWS_PER_TILE = N_ROWS // NW
CHUNK = 128
K = ROWS_PER_TILE // CHUNK

TC_B = 32


def _sc_gather_body(ids_hbm, table_hbm, out_hbm,
                    idx_v, buf_a, buf_b, gsem_a, gsem_b, osem_a, osem_b):
    wid = lax.axis_index("s") * NC + lax.axis_index("c")
    base_row = wid * ROWS_PER_TILE

    pltpu.sync_copy(ids_hbm.at[wid], idx_v)

    def fire_gather(k, buf, sem):
        pltpu.async_copy(table_hbm.at[idx_v.at[k]], buf, sem)

    def wait_gather(k, buf, sem):
        pltpu.make_async_copy(table_hbm.at[idx_v.at[k]], buf, sem).wait()

    def fire_scatter(k, buf, sem):
        pltpu.async_copy(buf, out_hbm.at[pl.ds(base_row + k * CHUNK, CHUNK)], sem)

    def wait_scatter(k, buf, sem):
        pltpu.make_async_copy(
            buf, out_hbm.at[pl.ds(base_row + k * CHUNK, CHUNK)], sem).wait()

    fire_gather(0, buf_a, gsem_a)
    fire_gather(1, buf_b, gsem_b)

    @pl.loop(0, K, step=2)
    def _chunk(k):
        wait_gather(k, buf_a, gsem_a)
        fire_scatter(k, buf_a, osem_a)

        wait_gather(k + 1, buf_b, gsem_b)
        fire_scatter(k + 1, buf_b, osem_b)

        wait_scatter(k, buf_a, osem_a)

        @pl.when(k + 2 < K)
        def _():
            fire_gather(k + 2, buf_a, gsem_a)

        wait_scatter(k + 1, buf_b, osem_b)

        @pl.when(k + 3 < K)
        def _():
            fire_gather(k + 3, buf_b, gsem_b)


def _tc_ln_body(x_ref, pos_ref, g_ref, b_ref, o_ref):
    t = x_ref[...] + pos_ref[...][None, :, :]
    mean = jnp.mean(t, axis=-1, keepdims=True)
    c = t - mean
    var = jnp.mean(c * c, axis=-1, keepdims=True)
    rstd = lax.rsqrt(var + EPS)
    o_ref[...] = c * rstd * g_ref[...] + b_ref[...]


@jax.jit
def _run(ids3d, token_table, pos_table, gamma, beta):
    mesh = plsc.VectorSubcoreMesh(core_axis_name="c", subcore_axis_name="s",
                                  num_cores=NC, num_subcores=NS)
    gathered = pl.kernel(
        _sc_gather_body,
        out_type=jax.ShapeDtypeStruct((N_ROWS, EMBED), jnp.float32),
        mesh=mesh,
        scratch_types=[
            pltpu.VMEM((K, CHUNK), jnp.int32),
            pltpu.VMEM((CHUNK, EMBED), jnp.float32),
            pltpu.VMEM((CHUNK, EMBED), jnp.float32),
            pltpu.SemaphoreType.DMA,
            pltpu.SemaphoreType.DMA,
            pltpu.SemaphoreType.DMA,
            pltpu.SemaphoreType.DMA,
        ],
    )(ids3d, token_table)

    x = gathered.reshape(BATCH, SEQ_LEN, EMBED)
    out = pl.pallas_call(
        _tc_ln_body,
        out_shape=jax.ShapeDtypeStruct((BATCH, SEQ_LEN, EMBED), jnp.float32),
        grid=(BATCH // TC_B,),
        in_specs=[
            pl.BlockSpec((TC_B, SEQ_LEN, EMBED), lambda i: (i, 0, 0)),
            pl.BlockSpec((SEQ_LEN, EMBED), lambda i: (0, 0)),
            pl.BlockSpec((EMBED,), lambda i: (0,)),
            pl.BlockSpec((EMBED,), lambda i: (0,)),
        ],
        out_specs=pl.BlockSpec((TC_B, SEQ_LEN, EMBED), lambda i: (i, 0, 0)),
    )(x, pos_table, gamma, beta)
    return out


def kernel(input_ids, token_table, pos_table, gamma, beta):
    ids3d = jnp.reshape(input_ids.astype(jnp.int32), (NW, K, CHUNK))
    return _run(ids3d, token_table, pos_table, gamma, beta)


# speedup vs baseline: 5.3075x; 1.0456x over previous
import jax
import jax.numpy as jnp
from jax import lax
from jax.experimental import pallas as pl
from jax.experimental.pallas import tpu as pltpu
from jax.experimental.pallas import tpu_sc as plsc

VOCAB = 100000
SEQ_LEN = 200
EMBED = 128
BATCH = 1024
EPS = 1e-5

NC = 2
NS = 16
NW = NC * NS
N_ROWS = BATCH * SEQ_LEN
ROWS_PER_TILE = N_ROWS // NW
CHUNK = 128
K = ROWS_PER_TILE // CHUNK

TC_B = 64


def _sc_gather_body(ids_hbm, table_hbm, out_hbm,
                    idx_v, buf_a, buf_b, gsem_a, gsem_b, osem_a, osem_b):
    wid = lax.axis_index("s") * NC + lax.axis_index("c")
    base_row = wid * ROWS_PER_TILE

    pltpu.sync_copy(ids_hbm.at[wid], idx_v)

    def fire_gather(k, buf, sem):
        pltpu.async_copy(table_hbm.at[idx_v.at[k]], buf, sem)

    def wait_gather(k, buf, sem):
        pltpu.make_async_copy(table_hbm.at[idx_v.at[k]], buf, sem).wait()

    def fire_scatter(k, buf, sem):
        pltpu.async_copy(buf, out_hbm.at[pl.ds(base_row + k * CHUNK, CHUNK)], sem)

    def wait_scatter(k, buf, sem):
        pltpu.make_async_copy(
            buf, out_hbm.at[pl.ds(base_row + k * CHUNK, CHUNK)], sem).wait()

    fire_gather(0, buf_a, gsem_a)
    fire_gather(1, buf_b, gsem_b)

    @pl.loop(0, K, step=2)
    def _chunk(k):
        wait_gather(k, buf_a, gsem_a)
        fire_scatter(k, buf_a, osem_a)

        wait_gather(k + 1, buf_b, gsem_b)
        fire_scatter(k + 1, buf_b, osem_b)

        wait_scatter(k, buf_a, osem_a)

        @pl.when(k + 2 < K)
        def _():
            fire_gather(k + 2, buf_a, gsem_a)

        wait_scatter(k + 1, buf_b, osem_b)

        @pl.when(k + 3 < K)
        def _():
            fire_gather(k + 3, buf_b, gsem_b)


def _tc_ln_body(x_ref, pos_ref, g_ref, b_ref, o_ref):
    t = x_ref[...] + pos_ref[...][None, :, :]
    mean = jnp.mean(t, axis=-1, keepdims=True)
    c = t - mean
    var = jnp.mean(c * c, axis=-1, keepdims=True)
    rstd = lax.rsqrt(var + EPS)
    o_ref[...] = c * rstd * g_ref[...] + b_ref[...]


@jax.jit
def _run(ids3d, token_table, pos_table, gamma, beta):
    mesh = plsc.VectorSubcoreMesh(core_axis_name="c", subcore_axis_name="s",
                                  num_cores=NC, num_subcores=NS)
    gathered = pl.kernel(
        _sc_gather_body,
        out_type=jax.ShapeDtypeStruct((N_ROWS, EMBED), jnp.float32),
        mesh=mesh,
        scratch_types=[
            pltpu.VMEM((K, CHUNK), jnp.int32),
            pltpu.VMEM((CHUNK, EMBED), jnp.float32),
            pltpu.VMEM((CHUNK, EMBED), jnp.float32),
            pltpu.SemaphoreType.DMA,
            pltpu.SemaphoreType.DMA,
            pltpu.SemaphoreType.DMA,
            pltpu.SemaphoreType.DMA,
        ],
    )(ids3d, token_table)

    x = gathered.reshape(BATCH, SEQ_LEN, EMBED)
    out = pl.pallas_call(
        _tc_ln_body,
        out_shape=jax.ShapeDtypeStruct((BATCH, SEQ_LEN, EMBED), jnp.float32),
        grid=(BATCH // TC_B,),
        in_specs=[
            pl.BlockSpec((TC_B, SEQ_LEN, EMBED), lambda i: (i, 0, 0)),
            pl.BlockSpec((SEQ_LEN, EMBED), lambda i: (0, 0)),
            pl.BlockSpec((EMBED,), lambda i: (0,)),
            pl.BlockSpec((EMBED,), lambda i: (0,)),
        ],
        out_specs=pl.BlockSpec((TC_B, SEQ_LEN, EMBED), lambda i: (i, 0, 0)),
    )(x, pos_table, gamma, beta)
    return out


def kernel(input_ids, token_table, pos_table, gamma, beta):
    ids3d = jnp.reshape(input_ids.astype(jnp.int32), (NW, K, CHUNK))
    return _run(ids3d, token_table, pos_table, gamma, beta)
